# baseline (speedup 1.0000x reference)
import functools

import jax
import jax.numpy as jnp
from jax import lax
from jax.experimental import pallas as pl
from jax.experimental.pallas import tpu as pltpu
from jax.experimental.pallas import tpu_sc as plsc

N = 10000
E = 160000
NC = 2
NS = 16
NW = NC * NS
K = 64
NB = 80
EPW = NB * K
EPAD = NW * EPW
NA = N + 8
NPT = 624
NPT_LAST = N - 15 * NPT


def _make_segsum(C, Wb):
  mesh = plsc.VectorSubcoreMesh(
      core_axis_name="c", subcore_axis_name="s", num_cores=NC, num_subcores=NS)

  @functools.partial(
      pl.kernel,
      out_type=jax.ShapeDtypeStruct((NC, C, N, Wb), jnp.float32),
      mesh=mesh,
      scratch_types=[
          pltpu.VMEM((NB, K), jnp.int32),
          pltpu.VMEM((NB, K), jnp.int32),
          pltpu.VMEM((3, K, Wb), jnp.float32),
          pltpu.VMEM_SHARED((NA, Wb), jnp.float32),
          pltpu.SemaphoreType.DMA,
          pltpu.SemaphoreType.DMA,
      ],
  )
  def seg(table, srcw, dstw, zeros, out, src_v, dst_v, buf, acc, gsem, ssem):
    c = lax.axis_index("c")
    s = lax.axis_index("s")
    w = s * NC + c
    pltpu.sync_copy(srcw.at[w], src_v)
    pltpu.sync_copy(dstw.at[w], dst_v)

    def per_f(f, carry):
      @pl.when(s < NS - 1)
      def _():
        pltpu.sync_copy(zeros.at[pl.ds(0, NPT)], acc.at[pl.ds(s * NPT, NPT)])

      @pl.when(s == NS - 1)
      def _():
        pltpu.sync_copy(zeros, acc.at[pl.ds(15 * NPT, NPT_LAST + 8)])

      plsc.subcore_barrier()

      for pb in range(2):
        pltpu.async_copy(table.at[f].at[src_v.at[pb]], buf.at[pb], gsem)

      def per_b(b, carry2):
        slot = lax.rem(b, 3)
        pltpu.make_async_copy(
            table.at[f].at[src_v.at[b]], buf.at[slot], gsem).wait()

        @pl.when(b + 2 < NB)
        def _():
          @pl.when(b >= 1)
          def _():
            pltpu.make_async_copy(
                buf.at[lax.rem(b - 1, 3)], acc.at[dst_v.at[b - 1]],
                ssem).wait()

          pltpu.async_copy(
              table.at[f].at[src_v.at[b + 2]], buf.at[lax.rem(b + 2, 3)],
              gsem)

        pltpu.make_async_copy(
            buf.at[slot], acc.at[dst_v.at[b]], ssem).start(add=True)
        return carry2

      lax.fori_loop(0, NB, per_b, 0)
      for t in range(NB - 3, NB):
        pltpu.make_async_copy(
            buf.at[t % 3], acc.at[dst_v.at[t]], ssem).wait()
      plsc.subcore_barrier()

      @pl.when(s < NS - 1)
      def _():
        pltpu.sync_copy(acc.at[pl.ds(s * NPT, NPT)],
                        out.at[c, f, pl.ds(s * NPT, NPT)])

      @pl.when(s == NS - 1)
      def _():
        pltpu.sync_copy(acc.at[pl.ds(15 * NPT, NPT_LAST)],
                        out.at[c, f, pl.ds(15 * NPT, NPT_LAST)])

      plsc.subcore_barrier()
      return carry

    lax.fori_loop(0, C, per_f, 0)

  return seg


def _elu(x):
  return jnp.where(x > 0, x, jnp.exp(jnp.minimum(x, 0.0)) - 1.0)


NB0 = 1000
NB2 = 200


def _tc0_body(degp, xi, mask, dinv, y0):
  deg = degp[0, 0][:, 0:1] + degp[1, 0][:, 0:1] + 1.0
  di = lax.rsqrt(deg)
  dinv[...] = di
  x = xi[...]
  y0[0] = di * x
  y0[1] = di * x * mask[...]


def _tc0(degp, xi, mask):
  g = N // NB0
  return pl.pallas_call(
      _tc0_body,
      grid=(g,),
      in_specs=[
          pl.BlockSpec((NC, 1, NB0, 128), lambda n: (0, 0, n, 0)),
          pl.BlockSpec((NB0, 128), lambda n: (n, 0)),
          pl.BlockSpec((NB0, 128), lambda n: (n, 0)),
      ],
      out_specs=[
          pl.BlockSpec((NB0, 1), lambda n: (n, 0)),
          pl.BlockSpec((2, NB0, 128), lambda n: (0, n, 0)),
      ],
      out_shape=[
          jax.ShapeDtypeStruct((N, 1), jnp.float32),
          jax.ShapeDtypeStruct((2, N, 128), jnp.float32),
      ],
  )(degp, xi, mask)


def _tc1_body(P, y0b, dinv, W1, b1, Z):
  di = dinv[...]
  p = di * (P[0, 0] + P[1, 0] + y0b[...])
  h = _elu(jnp.dot(p, W1[...], preferred_element_type=jnp.float32) + b1[...])
  z = di * h
  for f in range(16):
    Z[f] = z[:, f * 128:(f + 1) * 128]


def _tc1(P, y0b, dinv, W1, b1):
  g = N // NB0
  return pl.pallas_call(
      _tc1_body,
      grid=(g,),
      in_specs=[
          pl.BlockSpec((NC, 1, NB0, 128), lambda n: (0, 0, n, 0)),
          pl.BlockSpec((NB0, 128), lambda n: (n, 0)),
          pl.BlockSpec((NB0, 1), lambda n: (n, 0)),
          pl.BlockSpec((128, 2048), lambda n: (0, 0)),
          pl.BlockSpec((1, 2048), lambda n: (0, 0)),
      ],
      out_specs=pl.BlockSpec((16, NB0, 128), lambda n: (0, n, 0)),
      out_shape=jax.ShapeDtypeStruct((16, N, 128), jnp.float32),
  )(P, y0b, dinv, W1, b1)


def _tc2_body(P, z1, dinv, W2, b2, Wec, H, U, V):
  di = dinv[...]
  p = jnp.concatenate(
      [di * (P[0, f] + P[1, f] + z1[f]) for f in range(16)], axis=1)
  h = _elu(jnp.dot(p, W2[...], preferred_element_type=jnp.float32) + b2[...])
  H[...] = h
  uv = di * jnp.dot(h, Wec[...], preferred_element_type=jnp.float32)
  U[0] = uv[:, 0:128]
  U[1] = uv[:, 128:256]
  V[...] = jnp.concatenate(
      [uv[:, 256:272], jnp.zeros((uv.shape[0], 112), jnp.float32)], axis=1)


def _tc2(P, z1, dinv, W2, b2, Wec):
  g = N // NB2
  return pl.pallas_call(
      _tc2_body,
      grid=(g,),
      in_specs=[
          pl.BlockSpec((NC, 16, NB2, 128), lambda n: (0, 0, n, 0)),
          pl.BlockSpec((16, NB2, 128), lambda n: (0, n, 0)),
          pl.BlockSpec((NB2, 1), lambda n: (n, 0)),
          pl.BlockSpec((2048, 4096), lambda n: (0, 0)),
          pl.BlockSpec((1, 4096), lambda n: (0, 0)),
          pl.BlockSpec((4096, 272), lambda n: (0, 0)),
      ],
      out_specs=[
          pl.BlockSpec((NB2, 4096), lambda n: (n, 0)),
          pl.BlockSpec((2, NB2, 128), lambda n: (0, n, 0)),
          pl.BlockSpec((NB2, 128), lambda n: (n, 0)),
      ],
      out_shape=[
          jax.ShapeDtypeStruct((N, 4096), jnp.float32),
          jax.ShapeDtypeStruct((2, N, 128), jnp.float32),
          jax.ShapeDtypeStruct((N, 128), jnp.float32),
      ],
  )(P, z1, dinv, W2, b2, Wec)


def _softmax16(q):
  m = jnp.max(q, axis=1, keepdims=True)
  e = jnp.exp(q - m)
  return e / jnp.sum(e, axis=1, keepdims=True)


def _tc3a_body(Pu, U0, dinv, be, Wt):
  di = dinv[...]
  for c in range(2):
    pe = di * (Pu[0, c] + Pu[1, c] + U0[c]) + be[:, c * 128:(c + 1) * 128]
    Wt[c] = di * _elu(pe)


def _tc3a(Pu, U0, dinv, be):
  g = N // NB0
  return pl.pallas_call(
      _tc3a_body,
      grid=(g,),
      in_specs=[
          pl.BlockSpec((NC, 2, NB0, 128), lambda n: (0, 0, n, 0)),
          pl.BlockSpec((2, NB0, 128), lambda n: (0, n, 0)),
          pl.BlockSpec((NB0, 1), lambda n: (n, 0)),
          pl.BlockSpec((1, 256), lambda n: (0, 0)),
      ],
      out_specs=pl.BlockSpec((2, NB0, 128), lambda n: (0, n, 0)),
      out_shape=jax.ShapeDtypeStruct((2, N, 128), jnp.float32),
  )(Pu, U0, dinv, be)


def _tc3b_body(Pv, Vp, dinv, bp, ci, cj):
  q = dinv[...] * (Pv[0, 0] + Pv[1, 0] + Vp[...])
  ci[...] = _softmax16(q[:, 0:16] + bp[...])
  cj[...] = _softmax16(q[:, 16:32] + bp[...])


def _tc3b(Pv, Vp, dinv, bp):
  g = N // NB0
  return pl.pallas_call(
      _tc3b_body,
      grid=(g,),
      in_specs=[
          pl.BlockSpec((NC, 1, NB0, 128), lambda n: (0, 0, n, 0)),
          pl.BlockSpec((NB0, 128), lambda n: (n, 0)),
          pl.BlockSpec((NB0, 1), lambda n: (n, 0)),
          pl.BlockSpec((1, 16), lambda n: (0, 0)),
      ],
      out_specs=[
          pl.BlockSpec((NB0, 16), lambda n: (n, 0)),
          pl.BlockSpec((NB0, 16), lambda n: (n, 0)),
      ],
      out_shape=[
          jax.ShapeDtypeStruct((N, 16), jnp.float32),
          jax.ShapeDtypeStruct((N, 16), jnp.float32),
      ],
  )(Pv, Vp, dinv, bp)


def _tc4_body(Pw, Wt, dinv, W3, b3, W4, X4):
  di = dinv[...]
  pw = jnp.concatenate(
      [di * (Pw[0, c] + Pw[1, c] + Wt[c]) for c in range(2)], axis=1)
  up1 = _elu(jnp.dot(pw, W3[...], preferred_element_type=jnp.float32)
             + b3[...])
  t = jnp.dot(up1, W4[...], preferred_element_type=jnp.float32)
  X4[0] = di * t


def _tc4(Pw, Wt, dinv, W3, b3, W4):
  g = N // NB0
  return pl.pallas_call(
      _tc4_body,
      grid=(g,),
      in_specs=[
          pl.BlockSpec((NC, 2, NB0, 128), lambda n: (0, 0, n, 0)),
          pl.BlockSpec((2, NB0, 128), lambda n: (0, n, 0)),
          pl.BlockSpec((NB0, 1), lambda n: (n, 0)),
          pl.BlockSpec((256, 512), lambda n: (0, 0)),
          pl.BlockSpec((1, 512), lambda n: (0, 0)),
          pl.BlockSpec((512, 128), lambda n: (0, 0)),
      ],
      out_specs=pl.BlockSpec((1, NB0, 128), lambda n: (0, n, 0)),
      out_shape=jax.ShapeDtypeStruct((1, N, 128), jnp.float32),
  )(Pw, Wt, dinv, W3, b3, W4)


def _tc5_body(Px, X4, dinv, b4, up2):
  up2[...] = dinv[...] * (Px[0, 0] + Px[1, 0] + X4[0]) + b4[...]


def _tc5(Px, X4, dinv, b4):
  g = N // NB0
  return pl.pallas_call(
      _tc5_body,
      grid=(g,),
      in_specs=[
          pl.BlockSpec((NC, 1, NB0, 128), lambda n: (0, 0, n, 0)),
          pl.BlockSpec((1, NB0, 128), lambda n: (0, n, 0)),
          pl.BlockSpec((NB0, 1), lambda n: (n, 0)),
          pl.BlockSpec((1, 128), lambda n: (0, 0)),
      ],
      out_specs=pl.BlockSpec((NB0, 128), lambda n: (n, 0)),
      out_shape=jax.ShapeDtypeStruct((N, 128), jnp.float32),
  )(Px, X4, dinv, b4)


def kernel(xi, edge_index, mask, W1, b1, W2, b2, We, be, W3, b3, W4, b4,
           Wp, bp):
  pad = EPAD - E
  srcw = jnp.concatenate(
      [edge_index[0], jnp.zeros((pad,), jnp.int32)]).reshape(NW, NB, K)
  dstw = jnp.concatenate(
      [edge_index[1], jnp.full((pad,), N, jnp.int32)]).reshape(NW, NB, K)
  z128 = jnp.zeros((NPT_LAST + 8, 128), jnp.float32)
  ones_t = jnp.ones((1, N, 128), jnp.float32)
  Wec = jnp.concatenate([We, Wp], axis=1)
  b1r = b1.reshape(1, -1)
  b2r = b2.reshape(1, -1)
  ber = be.reshape(1, -1)
  b3r = b3.reshape(1, -1)
  b4r = b4.reshape(1, -1)
  bpr = bp.reshape(1, -1)

  seg_2_128 = _make_segsum(2, 128)
  seg_16_128 = _make_segsum(16, 128)
  seg_1_128 = _make_segsum(1, 128)

  degp = seg_1_128(ones_t, srcw, dstw, z128)
  dinv, y0 = _tc0(degp, xi, mask)

  P0i = seg_1_128(y0[0:1], srcw, dstw, z128)
  P0j = seg_1_128(y0[1:2], srcw, dstw, z128)
  Zi = _tc1(P0i, y0[0], dinv, W1, b1r)
  Zj = _tc1(P0j, y0[1], dinv, W1, b1r)

  P1i = seg_16_128(Zi, srcw, dstw, z128)
  P1j = seg_16_128(Zj, srcw, dstw, z128)
  hi2, U0, Vi = _tc2(P1i, Zi, dinv, W2, b2r, Wec)
  hj2, _, Vj = _tc2(P1j, Zj, dinv, W2, b2r, Wec)

  Pu = seg_2_128(U0, srcw, dstw, z128)
  Vp = jnp.concatenate(
      [Vi[:, 0:16], Vj[:, 0:16], jnp.zeros((N, 96), jnp.float32)],
      axis=1)
  Pv = seg_1_128(Vp[None], srcw, dstw, z128)
  Wt = _tc3a(Pu, U0, dinv, ber)

  Pw = seg_2_128(Wt, srcw, dstw, z128)
  ci, cj = _tc3b(Pv, Vp, dinv, bpr)
  X4 = _tc4(Pw, Wt, dinv, W3, b3r, W4)

  Px = seg_1_128(X4, srcw, dstw, z128)
  up2 = _tc5(Px, X4, dinv, b4r)

  return (hi2, hj2, ci, cj, up2)


# speedup vs baseline: 3.0068x; 3.0068x over previous
import functools

import jax
import jax.numpy as jnp
from jax import lax
from jax.experimental import pallas as pl
from jax.experimental.pallas import tpu as pltpu
from jax.experimental.pallas import tpu_sc as plsc

N = 10000
E = 160000
NC = 2
NS = 16
NW = NC * NS
EPW = E // NW
K = 125
NB = EPW // K
NPT = 624
NPT_LAST = N - 15 * NPT


def _make_segsum(C, Wb):
  mesh = plsc.VectorSubcoreMesh(
      core_axis_name="c", subcore_axis_name="s", num_cores=NC, num_subcores=NS)

  @functools.partial(
      pl.kernel,
      out_type=jax.ShapeDtypeStruct((NC, C, N, Wb), jnp.float32),
      mesh=mesh,
      scratch_types=[
          pltpu.VMEM((NB, K), jnp.int32),
          pltpu.VMEM((NB, K), jnp.int32),
          pltpu.VMEM((2, K, Wb), jnp.float32),
          pltpu.VMEM_SHARED((N, Wb), jnp.float32),
          pltpu.SemaphoreType.DMA,
          pltpu.SemaphoreType.DMA,
      ],
  )
  def seg(table, srcw, dstw, zeros, out, src_v, dst_v, buf, acc, gsem, ssem):
    c = lax.axis_index("c")
    s = lax.axis_index("s")
    w = s * NC + c
    pltpu.sync_copy(srcw.at[w], src_v)
    pltpu.sync_copy(dstw.at[w], dst_v)

    def per_f(f, carry):
      @pl.when(s < NS - 1)
      def _():
        pltpu.sync_copy(zeros.at[pl.ds(0, NPT)], acc.at[pl.ds(s * NPT, NPT)])

      @pl.when(s == NS - 1)
      def _():
        pltpu.sync_copy(zeros, acc.at[pl.ds(15 * NPT, NPT_LAST)])

      plsc.subcore_barrier()

      pltpu.async_copy(table.at[f].at[src_v.at[0]], buf.at[0], gsem)

      def per_b(b, carry2):
        slot = lax.rem(b, 2)
        pltpu.make_async_copy(
            table.at[f].at[src_v.at[b]], buf.at[slot], gsem).wait()

        @pl.when(b >= 1)
        def _():
          pltpu.make_async_copy(
              buf.at[1 - slot], acc.at[dst_v.at[b - 1]], ssem).wait()

        @pl.when(b + 1 < NB)
        def _():
          pltpu.async_copy(
              table.at[f].at[src_v.at[b + 1]], buf.at[1 - slot], gsem)

        pltpu.make_async_copy(
            buf.at[slot], acc.at[dst_v.at[b]], ssem).start(add=True)
        return carry2

      lax.fori_loop(0, NB, per_b, 0)
      pltpu.make_async_copy(
          buf.at[(NB - 1) % 2], acc.at[dst_v.at[NB - 1]], ssem).wait()
      plsc.subcore_barrier()

      @pl.when(s < NS - 1)
      def _():
        pltpu.sync_copy(acc.at[pl.ds(s * NPT, NPT)],
                        out.at[c, f, pl.ds(s * NPT, NPT)])

      @pl.when(s == NS - 1)
      def _():
        pltpu.sync_copy(acc.at[pl.ds(15 * NPT, NPT_LAST)],
                        out.at[c, f, pl.ds(15 * NPT, NPT_LAST)])

      plsc.subcore_barrier()
      return carry

    lax.fori_loop(0, C, per_f, 0)

  return seg


def _elu(x):
  return jnp.where(x > 0, x, jnp.exp(jnp.minimum(x, 0.0)) - 1.0)


NB0 = 1000
NB2 = 200


def _tc0_body(degp, xi, mask, dinv, y0):
  deg = degp[0, 0][:, 0:1] + degp[1, 0][:, 0:1] + 1.0
  di = lax.rsqrt(deg)
  dinv[...] = di
  x = xi[...]
  y0[0] = di * x
  y0[1] = di * x * mask[...]


def _tc0(degp, xi, mask):
  g = N // NB0
  return pl.pallas_call(
      _tc0_body,
      grid=(g,),
      in_specs=[
          pl.BlockSpec((NC, 1, NB0, 128), lambda n: (0, 0, n, 0)),
          pl.BlockSpec((NB0, 128), lambda n: (n, 0)),
          pl.BlockSpec((NB0, 128), lambda n: (n, 0)),
      ],
      out_specs=[
          pl.BlockSpec((NB0, 1), lambda n: (n, 0)),
          pl.BlockSpec((2, NB0, 128), lambda n: (0, n, 0)),
      ],
      out_shape=[
          jax.ShapeDtypeStruct((N, 1), jnp.float32),
          jax.ShapeDtypeStruct((2, N, 128), jnp.float32),
      ],
  )(degp, xi, mask)


def _tc1_body(P, y0b, dinv, W1, b1, Z):
  di = dinv[...]
  p = di * (P[0, 0] + P[1, 0] + y0b[...])
  h = _elu(jnp.dot(p, W1[...], preferred_element_type=jnp.float32) + b1[...])
  z = di * h
  for f in range(16):
    Z[f] = z[:, f * 128:(f + 1) * 128]


def _tc1(P, y0b, dinv, W1, b1):
  g = N // NB0
  return pl.pallas_call(
      _tc1_body,
      grid=(g,),
      in_specs=[
          pl.BlockSpec((NC, 1, NB0, 128), lambda n: (0, 0, n, 0)),
          pl.BlockSpec((NB0, 128), lambda n: (n, 0)),
          pl.BlockSpec((NB0, 1), lambda n: (n, 0)),
          pl.BlockSpec((128, 2048), lambda n: (0, 0)),
          pl.BlockSpec((1, 2048), lambda n: (0, 0)),
      ],
      out_specs=pl.BlockSpec((16, NB0, 128), lambda n: (0, n, 0)),
      out_shape=jax.ShapeDtypeStruct((16, N, 128), jnp.float32),
  )(P, y0b, dinv, W1, b1)


def _tc2_body(P, z1, dinv, W2, b2, Wec, H, U, V):
  di = dinv[...]
  p = jnp.concatenate(
      [di * (P[0, f] + P[1, f] + z1[f]) for f in range(16)], axis=1)
  h = _elu(jnp.dot(p, W2[...], preferred_element_type=jnp.float32) + b2[...])
  H[...] = h
  uv = di * jnp.dot(h, Wec[...], preferred_element_type=jnp.float32)
  U[0] = uv[:, 0:128]
  U[1] = uv[:, 128:256]
  V[...] = jnp.concatenate(
      [uv[:, 256:272], jnp.zeros((uv.shape[0], 112), jnp.float32)], axis=1)


def _tc2(P, z1, dinv, W2, b2, Wec):
  g = N // NB2
  return pl.pallas_call(
      _tc2_body,
      grid=(g,),
      in_specs=[
          pl.BlockSpec((NC, 16, NB2, 128), lambda n: (0, 0, n, 0)),
          pl.BlockSpec((16, NB2, 128), lambda n: (0, n, 0)),
          pl.BlockSpec((NB2, 1), lambda n: (n, 0)),
          pl.BlockSpec((2048, 4096), lambda n: (0, 0)),
          pl.BlockSpec((1, 4096), lambda n: (0, 0)),
          pl.BlockSpec((4096, 272), lambda n: (0, 0)),
      ],
      out_specs=[
          pl.BlockSpec((NB2, 4096), lambda n: (n, 0)),
          pl.BlockSpec((2, NB2, 128), lambda n: (0, n, 0)),
          pl.BlockSpec((NB2, 128), lambda n: (n, 0)),
      ],
      out_shape=[
          jax.ShapeDtypeStruct((N, 4096), jnp.float32),
          jax.ShapeDtypeStruct((2, N, 128), jnp.float32),
          jax.ShapeDtypeStruct((N, 128), jnp.float32),
      ],
  )(P, z1, dinv, W2, b2, Wec)


def _softmax16(q):
  m = jnp.max(q, axis=1, keepdims=True)
  e = jnp.exp(q - m)
  return e / jnp.sum(e, axis=1, keepdims=True)


def _tc3a_body(Pu, U0, dinv, be, Wt):
  di = dinv[...]
  for c in range(2):
    pe = di * (Pu[0, c] + Pu[1, c] + U0[c]) + be[:, c * 128:(c + 1) * 128]
    Wt[c] = di * _elu(pe)


def _tc3a(Pu, U0, dinv, be):
  g = N // NB0
  return pl.pallas_call(
      _tc3a_body,
      grid=(g,),
      in_specs=[
          pl.BlockSpec((NC, 2, NB0, 128), lambda n: (0, 0, n, 0)),
          pl.BlockSpec((2, NB0, 128), lambda n: (0, n, 0)),
          pl.BlockSpec((NB0, 1), lambda n: (n, 0)),
          pl.BlockSpec((1, 256), lambda n: (0, 0)),
      ],
      out_specs=pl.BlockSpec((2, NB0, 128), lambda n: (0, n, 0)),
      out_shape=jax.ShapeDtypeStruct((2, N, 128), jnp.float32),
  )(Pu, U0, dinv, be)


def _tc3b_body(Pv, Vp, dinv, bp, ci, cj):
  q = dinv[...] * (Pv[0, 0] + Pv[1, 0] + Vp[...])
  ci[...] = _softmax16(q[:, 0:16] + bp[...])
  cj[...] = _softmax16(q[:, 16:32] + bp[...])


def _tc3b(Pv, Vp, dinv, bp):
  g = N // NB0
  return pl.pallas_call(
      _tc3b_body,
      grid=(g,),
      in_specs=[
          pl.BlockSpec((NC, 1, NB0, 128), lambda n: (0, 0, n, 0)),
          pl.BlockSpec((NB0, 128), lambda n: (n, 0)),
          pl.BlockSpec((NB0, 1), lambda n: (n, 0)),
          pl.BlockSpec((1, 16), lambda n: (0, 0)),
      ],
      out_specs=[
          pl.BlockSpec((NB0, 16), lambda n: (n, 0)),
          pl.BlockSpec((NB0, 16), lambda n: (n, 0)),
      ],
      out_shape=[
          jax.ShapeDtypeStruct((N, 16), jnp.float32),
          jax.ShapeDtypeStruct((N, 16), jnp.float32),
      ],
  )(Pv, Vp, dinv, bp)


def _tc4_body(Pw, Wt, dinv, W3, b3, W4, X4):
  di = dinv[...]
  pw = jnp.concatenate(
      [di * (Pw[0, c] + Pw[1, c] + Wt[c]) for c in range(2)], axis=1)
  up1 = _elu(jnp.dot(pw, W3[...], preferred_element_type=jnp.float32)
             + b3[...])
  t = jnp.dot(up1, W4[...], preferred_element_type=jnp.float32)
  X4[0] = di * t


def _tc4(Pw, Wt, dinv, W3, b3, W4):
  g = N // NB0
  return pl.pallas_call(
      _tc4_body,
      grid=(g,),
      in_specs=[
          pl.BlockSpec((NC, 2, NB0, 128), lambda n: (0, 0, n, 0)),
          pl.BlockSpec((2, NB0, 128), lambda n: (0, n, 0)),
          pl.BlockSpec((NB0, 1), lambda n: (n, 0)),
          pl.BlockSpec((256, 512), lambda n: (0, 0)),
          pl.BlockSpec((1, 512), lambda n: (0, 0)),
          pl.BlockSpec((512, 128), lambda n: (0, 0)),
      ],
      out_specs=pl.BlockSpec((1, NB0, 128), lambda n: (0, n, 0)),
      out_shape=jax.ShapeDtypeStruct((1, N, 128), jnp.float32),
  )(Pw, Wt, dinv, W3, b3, W4)


def _tc5_body(Px, X4, dinv, b4, up2):
  up2[...] = dinv[...] * (Px[0, 0] + Px[1, 0] + X4[0]) + b4[...]


def _tc5(Px, X4, dinv, b4):
  g = N // NB0
  return pl.pallas_call(
      _tc5_body,
      grid=(g,),
      in_specs=[
          pl.BlockSpec((NC, 1, NB0, 128), lambda n: (0, 0, n, 0)),
          pl.BlockSpec((1, NB0, 128), lambda n: (0, n, 0)),
          pl.BlockSpec((NB0, 1), lambda n: (n, 0)),
          pl.BlockSpec((1, 128), lambda n: (0, 0)),
      ],
      out_specs=pl.BlockSpec((NB0, 128), lambda n: (n, 0)),
      out_shape=jax.ShapeDtypeStruct((N, 128), jnp.float32),
  )(Px, X4, dinv, b4)


def kernel(xi, edge_index, mask, W1, b1, W2, b2, We, be, W3, b3, W4, b4,
           Wp, bp):
  srcw = edge_index[0].reshape(NW, NB, K)
  dstw = edge_index[1].reshape(NW, NB, K)
  z128 = jnp.zeros((NPT_LAST, 128), jnp.float32)
  ones_t = jnp.ones((1, N, 128), jnp.float32)
  Wec = jnp.concatenate([We, Wp], axis=1)
  b1r = b1.reshape(1, -1)
  b2r = b2.reshape(1, -1)
  ber = be.reshape(1, -1)
  b3r = b3.reshape(1, -1)
  b4r = b4.reshape(1, -1)
  bpr = bp.reshape(1, -1)

  seg_2_128 = _make_segsum(2, 128)
  seg_16_128 = _make_segsum(16, 128)
  seg_1_128 = _make_segsum(1, 128)

  degp = seg_1_128(ones_t, srcw, dstw, z128)
  dinv, y0 = _tc0(degp, xi, mask)

  P0i = seg_1_128(y0[0:1], srcw, dstw, z128)
  P0j = seg_1_128(y0[1:2], srcw, dstw, z128)
  Zi = _tc1(P0i, y0[0], dinv, W1, b1r)
  Zj = _tc1(P0j, y0[1], dinv, W1, b1r)

  P1i = seg_16_128(Zi, srcw, dstw, z128)
  P1j = seg_16_128(Zj, srcw, dstw, z128)
  hi2, U0, Vi = _tc2(P1i, Zi, dinv, W2, b2r, Wec)
  hj2, _, Vj = _tc2(P1j, Zj, dinv, W2, b2r, Wec)

  Pu = seg_2_128(U0, srcw, dstw, z128)
  Vp = jnp.concatenate(
      [Vi[:, 0:16], Vj[:, 0:16], jnp.zeros((N, 96), jnp.float32)],
      axis=1)
  Pv = seg_1_128(Vp[None], srcw, dstw, z128)
  Wt = _tc3a(Pu, U0, dinv, ber)

  Pw = seg_2_128(Wt, srcw, dstw, z128)
  ci, cj = _tc3b(Pv, Vp, dinv, bpr)
  X4 = _tc4(Pw, Wt, dinv, W3, b3r, W4)

  Px = seg_1_128(X4, srcw, dstw, z128)
  up2 = _tc5(Px, X4, dinv, b4r)

  return (hi2, hj2, ci, cj, up2)
